# baseline (speedup 1.0000x reference)
import functools

import jax
import jax.numpy as jnp
from jax.experimental import pallas as pl
from jax.experimental.pallas import tpu as pltpu
from jax.experimental.pallas import tpu_sc as plsc

N = 8192
IN_CH = 64
OUT_CH = 64
K = 16
EXTENT = 0.1
KS = 4
S = KS * KS * KS

RB = 256
NBLK = N // RB
CW = 256

SC_NC = 2
SC_NS = 16
SC_NW = SC_NC * SC_NS
GCH = 128
TW = 80


def _atan_poly(t):
    a = jnp.abs(t)
    big = a > 0.4142135623730951
    x = jnp.where(big, (a - 1.0) / (a + 1.0), a)
    z = x * x
    p = (((8.05374449538e-2 * z - 1.38776856032e-1) * z + 1.99777106478e-1) * z
         - 3.33329491539e-1) * z * x + x
    p = jnp.where(big, 0.7853981633974483 + p, p)
    return jnp.sign(t) * p


def _knn_kernel(pos_ref, posT_ref, sq_ref, sqT_ref, idx_ref, d2_ref):
    pid = pl.program_id(0)
    pblk = pos_ref[...]
    sqb = sq_ref[...]
    rid = pid * RB + jax.lax.broadcasted_iota(jnp.int32, (RB, CW), 0)
    cid0 = jax.lax.broadcasted_iota(jnp.int32, (RB, CW), 1)

    def build(c, carry):
        off = pl.multiple_of(c * CW, CW)
        dot = jnp.dot(pblk, posT_ref[:, pl.ds(off, CW)],
                      preferred_element_type=jnp.float32)
        d2 = sqb + sqT_ref[0:1, pl.ds(off, CW)] - 2.0 * dot
        d2_ref[:, pl.ds(off, CW)] = jnp.where(rid == (cid0 + c * CW), jnp.inf, d2)
        return carry

    jax.lax.fori_loop(0, N // CW, build, 0)

    slot = jax.lax.broadcasted_iota(jnp.int32, (RB, K), 1)

    def outer(t, carry):
        lv, li, acc = carry

        def inner(c, ic):
            mv, mi = ic
            off = pl.multiple_of(c * CW, CW)
            d2c = d2_ref[:, pl.ds(off, CW)]
            colc = cid0 + c * CW
            pred = (d2c > lv) | ((d2c == lv) & (colc > li))
            cand = jnp.where(pred, d2c, jnp.inf)
            lmv = jnp.min(cand, axis=1, keepdims=True)
            lmi = jnp.min(jnp.where(cand == lmv, colc, N), axis=1, keepdims=True)
            better = (lmv < mv) | ((lmv == mv) & (lmi < mi))
            return jnp.where(better, lmv, mv), jnp.where(better, lmi, mi)

        mv0 = jnp.full((RB, 1), jnp.inf, jnp.float32)
        mi0 = jnp.full((RB, 1), N, jnp.int32)
        mv, mi = jax.lax.fori_loop(0, N // CW, inner, (mv0, mi0))
        acc = jnp.where(slot == t, mi, acc)
        return mv, mi, acc

    lv0 = jnp.full((RB, 1), -jnp.inf, jnp.float32)
    li0 = jnp.full((RB, 1), -1, jnp.int32)
    acc0 = jnp.zeros((RB, K), jnp.int32)
    _, _, acc = jax.lax.fori_loop(0, K, outer, (lv0, li0, acc0))
    idx_ref[...] = acc


def _knn(pos, posT, sq):
    return pl.pallas_call(
        _knn_kernel,
        grid=(NBLK,),
        in_specs=[
            pl.BlockSpec((RB, 3), lambda i: (i, 0)),
            pl.BlockSpec((3, N), lambda i: (0, 0)),
            pl.BlockSpec((RB, 1), lambda i: (i, 0)),
            pl.BlockSpec((1, N), lambda i: (0, 0)),
        ],
        out_specs=pl.BlockSpec((RB, K), lambda i: (i, 0)),
        out_shape=jax.ShapeDtypeStruct((N, K), jnp.int32),
        scratch_shapes=[pltpu.VMEM((RB, N), jnp.float32)],
    )(pos, posT, sq.reshape(N, 1), sq.reshape(1, N))


def _sc_gather(table, idx_flat):
    b_per_w = (N * K) // SC_NW
    nch = b_per_w // GCH
    mesh = plsc.VectorSubcoreMesh(core_axis_name="c", subcore_axis_name="s")

    @functools.partial(
        pl.kernel,
        mesh=mesh,
        compiler_params=pltpu.CompilerParams(use_tc_tiling_on_sc=False),
        out_type=jax.ShapeDtypeStruct((N * K, TW), jnp.float32),
        scratch_types=[
            pltpu.VMEM((GCH,), jnp.int32),
            pltpu.VMEM((GCH, TW), jnp.float32),
            pltpu.SemaphoreType.DMA,
        ],
    )
    def k(table_hbm, idx_hbm, out_hbm, idx_v, rows_v, sem):
        wid = jax.lax.axis_index("s") * SC_NC + jax.lax.axis_index("c")
        base = wid * b_per_w

        def body(c, carry):
            start = base + c * GCH
            pltpu.sync_copy(idx_hbm.at[pl.ds(start, GCH)], idx_v)
            pltpu.async_copy(table_hbm.at[idx_v], rows_v, sem).wait()
            pltpu.sync_copy(rows_v, out_hbm.at[pl.ds(start, GCH)])
            return carry

        jax.lax.fori_loop(0, nch, body, 0)

    return k(table, idx_flat)


def _conv_kernel(gath_ref, pos_ref, feats_ref, wfr_ref, bc_ref, wdt_ref,
                 bd_ref, conv_ref, dense_ref):
    g = gath_ref[...]
    nbf = g[:, :, 0:IN_CH]
    nbp = g[:, :, IN_CH:IN_CH + 3]
    p = pos_ref[...]
    rel = nbp - p[:, None, :]
    dist2 = jnp.sum(rel * rel, axis=2)
    radius = EXTENT / 2.0
    valid = (dist2 <= radius * radius).astype(jnp.float32)

    rel_n = rel * (2.0 / EXTENT)
    nrm = jnp.sqrt(jnp.sum(rel_n * rel_n, axis=2) + 1e-20)
    scale = jnp.minimum(1.0, 1.0 / nrm)
    x = rel_n[:, :, 0] * scale
    y = rel_n[:, :, 1] * scale
    z = rel_n[:, :, 2] * scale

    sq_norm = x * x + y * y + z * z
    norm = jnp.sqrt(sq_norm + 1e-20)
    sq_xy = x * x + y * y
    cond = (5.0 / 4.0) * z * z > sq_xy
    s1 = jnp.sqrt(3.0 * norm / (norm + jnp.abs(z) + 1e-20))
    x1, y1, z1 = x * s1, y * s1, jnp.sign(z) * norm
    s2 = norm / jnp.sqrt(sq_xy + 1e-20)
    x2, y2, z2 = x * s2, y * s2, 1.5 * z
    x = jnp.where(cond, x1, x2)
    y = jnp.where(cond, y1, y2)
    z = jnp.where(cond, z1, z2)
    nz = sq_norm > 1e-18
    x = jnp.where(nz, x, 0.0)
    y = jnp.where(nz, y, 0.0)
    z = jnp.where(nz, z, 0.0)
    sq_xy2 = x * x + y * y
    norm_xy = jnp.sqrt(sq_xy2 + 1e-20)
    cond2 = jnp.abs(y) <= jnp.abs(x)
    safe_x = jnp.where(jnp.abs(x) > 1e-12, x, 1.0)
    t1 = jnp.where(jnp.abs(x) > 1e-12, y / safe_x, 0.0)
    a1 = jnp.sign(x) * norm_xy
    b1 = jnp.sign(x) * norm_xy * (4.0 / jnp.pi) * _atan_poly(t1)
    safe_y = jnp.where(jnp.abs(y) > 1e-12, y, 1.0)
    t2 = jnp.where(jnp.abs(y) > 1e-12, x / safe_y, 0.0)
    b2 = jnp.sign(y) * norm_xy
    a2 = jnp.sign(y) * norm_xy * (4.0 / jnp.pi) * _atan_poly(t2)
    cx = jnp.where(cond2, a1, a2)
    cy = jnp.where(cond2, b1, b2)
    nz2 = sq_xy2 > 1e-18
    cx = jnp.where(nz2, cx, 0.0)
    cy = jnp.where(nz2, cy, 0.0)
    cz = z

    def axis_grid(cc):
        gg = (cc * 0.5 + 0.5) * (KS - 1)
        gg = jnp.clip(gg, 0.0, KS - 1.0)
        i0f = jnp.clip(jnp.floor(gg), 0.0, KS - 2.0)
        return i0f.astype(jnp.int32), gg - i0f

    i0x, fx = axis_grid(cx)
    i0y, fy = axis_grid(cy)
    i0z, fz = axis_grid(cz)

    sI = jax.lax.broadcasted_iota(jnp.int32, (RB3, K, S), 2)
    izI = sI // (KS * KS)
    iyI = (sI // KS) % KS
    ixI = sI % KS

    def axis_w(aI, i0, f):
        i0e = i0[:, :, None]
        fe = f[:, :, None]
        return (jnp.where(aI == i0e, 1.0 - fe, 0.0)
                + jnp.where(aI == i0e + 1, fe, 0.0))

    w3 = (axis_w(izI, i0z, fz) * axis_w(iyI, i0y, fy) * axis_w(ixI, i0x, fx)
          * valid[:, :, None])

    acc = jax.lax.dot_general(w3, nbf, (((1,), (1,)), ((0,), (0,))),
                              preferred_element_type=jnp.float32,
                              precision=jax.lax.Precision.HIGHEST)
    outS = jax.lax.dot_general(acc, wfr_ref[...], (((2,), (1,)), ((1,), (0,))),
                               preferred_element_type=jnp.float32)
    conv_ref[...] = jnp.sum(outS, axis=0) + bc_ref[...]

    dense_ref[...] = jnp.dot(feats_ref[...], wdt_ref[...],
                             preferred_element_type=jnp.float32) + bd_ref[...]


RB3 = 128
NBLK3 = N // RB3


def _conv(gath, pos, feats, wfr, bc, wdt, bd):
    return pl.pallas_call(
        _conv_kernel,
        grid=(NBLK3,),
        in_specs=[
            pl.BlockSpec((RB3, K, TW), lambda i: (i, 0, 0)),
            pl.BlockSpec((RB3, 3), lambda i: (i, 0)),
            pl.BlockSpec((RB3, IN_CH), lambda i: (i, 0)),
            pl.BlockSpec((S, IN_CH, OUT_CH), lambda i: (0, 0, 0)),
            pl.BlockSpec((1, OUT_CH), lambda i: (0, 0)),
            pl.BlockSpec((IN_CH, OUT_CH), lambda i: (0, 0)),
            pl.BlockSpec((1, OUT_CH), lambda i: (0, 0)),
        ],
        out_specs=[
            pl.BlockSpec((RB3, OUT_CH), lambda i: (i, 0)),
            pl.BlockSpec((RB3, OUT_CH), lambda i: (i, 0)),
        ],
        out_shape=[
            jax.ShapeDtypeStruct((N, OUT_CH), jnp.float32),
            jax.ShapeDtypeStruct((N, OUT_CH), jnp.float32),
        ],
    )(gath, pos, feats, wfr, bc, wdt, bd)


def kernel(feats, pos, W_conv, b_conv, W_dense, b_dense):
    posT = pos.T
    sq = jnp.sum(pos * pos, axis=1)
    idx = _knn(pos, posT, sq)
    table = jnp.concatenate(
        [feats, pos, jnp.zeros((N, TW - IN_CH - 3), jnp.float32)], axis=1)
    gath = _sc_gather(table, idx.reshape(N * K))
    wfr = W_conv.reshape(S, IN_CH, OUT_CH)
    ans_conv, ans_dense = _conv(
        gath.reshape(N, K, TW), pos, feats,
        wfr, b_conv.reshape(1, OUT_CH), W_dense.T, b_dense.reshape(1, OUT_CH))
    return ans_conv, ans_dense


# speedup vs baseline: 2.1776x; 2.1776x over previous
import functools

import jax
import jax.numpy as jnp
from jax.experimental import pallas as pl
from jax.experimental.pallas import tpu as pltpu
from jax.experimental.pallas import tpu_sc as plsc

N = 8192
IN_CH = 64
OUT_CH = 64
K = 16
EXTENT = 0.1
KS = 4
S = KS * KS * KS

RB = 256
NBLK = N // RB
CW = 4096

SC_NC = 2
SC_NS = 16
SC_NW = SC_NC * SC_NS
GCH = 128
TW = 80


def _atan_poly(t):
    a = jnp.abs(t)
    big = a > 0.4142135623730951
    x = jnp.where(big, (a - 1.0) / (a + 1.0), a)
    z = x * x
    p = (((8.05374449538e-2 * z - 1.38776856032e-1) * z + 1.99777106478e-1) * z
         - 3.33329491539e-1) * z * x + x
    p = jnp.where(big, 0.7853981633974483 + p, p)
    return jnp.sign(t) * p


def _knn_kernel(pos_ref, posT_ref, sq_ref, sqT_ref, idx_ref, d2_ref):
    pid = pl.program_id(0)
    pblk = pos_ref[...]
    sqb = sq_ref[...]
    rid = pid * RB + jax.lax.broadcasted_iota(jnp.int32, (RB, CW), 0)
    cid0 = jax.lax.broadcasted_iota(jnp.int32, (RB, CW), 1)

    def build(c, carry):
        off = pl.multiple_of(c * CW, CW)
        dot = jnp.dot(pblk, posT_ref[:, pl.ds(off, CW)],
                      preferred_element_type=jnp.float32)
        d2 = sqb + sqT_ref[0:1, pl.ds(off, CW)] - 2.0 * dot
        d2_ref[:, pl.ds(off, CW)] = jnp.where(rid == (cid0 + c * CW), jnp.inf, d2)
        return carry

    jax.lax.fori_loop(0, N // CW, build, 0)

    slot = jax.lax.broadcasted_iota(jnp.int32, (RB, K), 1)

    def outer(t, carry):
        lv, li, acc = carry

        def inner(c, ic):
            mv, mi = ic
            off = pl.multiple_of(c * CW, CW)
            d2c = d2_ref[:, pl.ds(off, CW)]
            colc = cid0 + c * CW
            pred = (d2c > lv) | ((d2c == lv) & (colc > li))
            cand = jnp.where(pred, d2c, jnp.inf)
            lmv = jnp.min(cand, axis=1, keepdims=True)
            lmi = jnp.min(jnp.where(cand == lmv, colc, N), axis=1, keepdims=True)
            better = (lmv < mv) | ((lmv == mv) & (lmi < mi))
            return jnp.where(better, lmv, mv), jnp.where(better, lmi, mi)

        mv0 = jnp.full((RB, 1), jnp.inf, jnp.float32)
        mi0 = jnp.full((RB, 1), N, jnp.int32)
        mv, mi = jax.lax.fori_loop(0, N // CW, inner, (mv0, mi0))
        acc = jnp.where(slot == t, mi, acc)
        return mv, mi, acc

    lv0 = jnp.full((RB, 1), -jnp.inf, jnp.float32)
    li0 = jnp.full((RB, 1), -1, jnp.int32)
    acc0 = jnp.zeros((RB, K), jnp.int32)
    _, _, acc = jax.lax.fori_loop(0, K, outer, (lv0, li0, acc0))
    idx_ref[...] = acc


def _knn(pos, posT, sq):
    return pl.pallas_call(
        _knn_kernel,
        grid=(NBLK,),
        in_specs=[
            pl.BlockSpec((RB, 3), lambda i: (i, 0)),
            pl.BlockSpec((3, N), lambda i: (0, 0)),
            pl.BlockSpec((RB, 1), lambda i: (i, 0)),
            pl.BlockSpec((1, N), lambda i: (0, 0)),
        ],
        out_specs=pl.BlockSpec((RB, K), lambda i: (i, 0)),
        out_shape=jax.ShapeDtypeStruct((N, K), jnp.int32),
        scratch_shapes=[pltpu.VMEM((RB, N), jnp.float32)],
    )(pos, posT, sq.reshape(N, 1), sq.reshape(1, N))


def _sc_gather(table, idx_flat):
    b_per_w = (N * K) // SC_NW
    nch = b_per_w // GCH
    mesh = plsc.VectorSubcoreMesh(core_axis_name="c", subcore_axis_name="s")

    @functools.partial(
        pl.kernel,
        mesh=mesh,
        compiler_params=pltpu.CompilerParams(use_tc_tiling_on_sc=False),
        out_type=jax.ShapeDtypeStruct((N * K, TW), jnp.float32),
        scratch_types=[
            pltpu.VMEM((GCH,), jnp.int32),
            pltpu.VMEM((GCH, TW), jnp.float32),
            pltpu.SemaphoreType.DMA,
        ],
    )
    def k(table_hbm, idx_hbm, out_hbm, idx_v, rows_v, sem):
        wid = jax.lax.axis_index("s") * SC_NC + jax.lax.axis_index("c")
        base = wid * b_per_w

        def body(c, carry):
            start = base + c * GCH
            pltpu.sync_copy(idx_hbm.at[pl.ds(start, GCH)], idx_v)
            pltpu.async_copy(table_hbm.at[idx_v], rows_v, sem).wait()
            pltpu.sync_copy(rows_v, out_hbm.at[pl.ds(start, GCH)])
            return carry

        jax.lax.fori_loop(0, nch, body, 0)

    return k(table, idx_flat)


def _conv_kernel(gath_ref, pos_ref, feats_ref, wfr_ref, bc_ref, wdt_ref,
                 bd_ref, conv_ref, dense_ref):
    g = gath_ref[...]
    nbf = g[:, :, 0:IN_CH]
    nbp = g[:, :, IN_CH:IN_CH + 3]
    p = pos_ref[...]
    rel = nbp - p[:, None, :]
    dist2 = jnp.sum(rel * rel, axis=2)
    radius = EXTENT / 2.0
    valid = (dist2 <= radius * radius).astype(jnp.float32)

    rel_n = rel * (2.0 / EXTENT)
    nrm = jnp.sqrt(jnp.sum(rel_n * rel_n, axis=2) + 1e-20)
    scale = jnp.minimum(1.0, 1.0 / nrm)
    x = rel_n[:, :, 0] * scale
    y = rel_n[:, :, 1] * scale
    z = rel_n[:, :, 2] * scale

    sq_norm = x * x + y * y + z * z
    norm = jnp.sqrt(sq_norm + 1e-20)
    sq_xy = x * x + y * y
    cond = (5.0 / 4.0) * z * z > sq_xy
    s1 = jnp.sqrt(3.0 * norm / (norm + jnp.abs(z) + 1e-20))
    x1, y1, z1 = x * s1, y * s1, jnp.sign(z) * norm
    s2 = norm / jnp.sqrt(sq_xy + 1e-20)
    x2, y2, z2 = x * s2, y * s2, 1.5 * z
    x = jnp.where(cond, x1, x2)
    y = jnp.where(cond, y1, y2)
    z = jnp.where(cond, z1, z2)
    nz = sq_norm > 1e-18
    x = jnp.where(nz, x, 0.0)
    y = jnp.where(nz, y, 0.0)
    z = jnp.where(nz, z, 0.0)
    sq_xy2 = x * x + y * y
    norm_xy = jnp.sqrt(sq_xy2 + 1e-20)
    cond2 = jnp.abs(y) <= jnp.abs(x)
    safe_x = jnp.where(jnp.abs(x) > 1e-12, x, 1.0)
    t1 = jnp.where(jnp.abs(x) > 1e-12, y / safe_x, 0.0)
    a1 = jnp.sign(x) * norm_xy
    b1 = jnp.sign(x) * norm_xy * (4.0 / jnp.pi) * _atan_poly(t1)
    safe_y = jnp.where(jnp.abs(y) > 1e-12, y, 1.0)
    t2 = jnp.where(jnp.abs(y) > 1e-12, x / safe_y, 0.0)
    b2 = jnp.sign(y) * norm_xy
    a2 = jnp.sign(y) * norm_xy * (4.0 / jnp.pi) * _atan_poly(t2)
    cx = jnp.where(cond2, a1, a2)
    cy = jnp.where(cond2, b1, b2)
    nz2 = sq_xy2 > 1e-18
    cx = jnp.where(nz2, cx, 0.0)
    cy = jnp.where(nz2, cy, 0.0)
    cz = z

    def axis_grid(cc):
        gg = (cc * 0.5 + 0.5) * (KS - 1)
        gg = jnp.clip(gg, 0.0, KS - 1.0)
        i0f = jnp.clip(jnp.floor(gg), 0.0, KS - 2.0)
        return i0f.astype(jnp.int32), gg - i0f

    i0x, fx = axis_grid(cx)
    i0y, fy = axis_grid(cy)
    i0z, fz = axis_grid(cz)

    sI = jax.lax.broadcasted_iota(jnp.int32, (RB3, K, S), 2)
    izI = sI // (KS * KS)
    iyI = (sI // KS) % KS
    ixI = sI % KS

    def axis_w(aI, i0, f):
        i0e = i0[:, :, None]
        fe = f[:, :, None]
        return (jnp.where(aI == i0e, 1.0 - fe, 0.0)
                + jnp.where(aI == i0e + 1, fe, 0.0))

    w3 = (axis_w(izI, i0z, fz) * axis_w(iyI, i0y, fy) * axis_w(ixI, i0x, fx)
          * valid[:, :, None])

    acc = jax.lax.dot_general(w3, nbf, (((1,), (1,)), ((0,), (0,))),
                              preferred_element_type=jnp.float32,
                              precision=jax.lax.Precision.HIGHEST)
    outS = jax.lax.dot_general(acc, wfr_ref[...], (((2,), (1,)), ((1,), (0,))),
                               preferred_element_type=jnp.float32)
    conv_ref[...] = jnp.sum(outS, axis=0) + bc_ref[...]

    dense_ref[...] = jnp.dot(feats_ref[...], wdt_ref[...],
                             preferred_element_type=jnp.float32) + bd_ref[...]


RB3 = 128
NBLK3 = N // RB3


def _conv(gath, pos, feats, wfr, bc, wdt, bd):
    return pl.pallas_call(
        _conv_kernel,
        grid=(NBLK3,),
        in_specs=[
            pl.BlockSpec((RB3, K, TW), lambda i: (i, 0, 0)),
            pl.BlockSpec((RB3, 3), lambda i: (i, 0)),
            pl.BlockSpec((RB3, IN_CH), lambda i: (i, 0)),
            pl.BlockSpec((S, IN_CH, OUT_CH), lambda i: (0, 0, 0)),
            pl.BlockSpec((1, OUT_CH), lambda i: (0, 0)),
            pl.BlockSpec((IN_CH, OUT_CH), lambda i: (0, 0)),
            pl.BlockSpec((1, OUT_CH), lambda i: (0, 0)),
        ],
        out_specs=[
            pl.BlockSpec((RB3, OUT_CH), lambda i: (i, 0)),
            pl.BlockSpec((RB3, OUT_CH), lambda i: (i, 0)),
        ],
        out_shape=[
            jax.ShapeDtypeStruct((N, OUT_CH), jnp.float32),
            jax.ShapeDtypeStruct((N, OUT_CH), jnp.float32),
        ],
    )(gath, pos, feats, wfr, bc, wdt, bd)


def kernel(feats, pos, W_conv, b_conv, W_dense, b_dense):
    posT = pos.T
    sq = jnp.sum(pos * pos, axis=1)
    idx = _knn(pos, posT, sq)
    table = jnp.concatenate(
        [feats, pos, jnp.zeros((N, TW - IN_CH - 3), jnp.float32)], axis=1)
    gath = _sc_gather(table, idx.reshape(N * K))
    wfr = W_conv.reshape(S, IN_CH, OUT_CH)
    ans_conv, ans_dense = _conv(
        gath.reshape(N, K, TW), pos, feats,
        wfr, b_conv.reshape(1, OUT_CH), W_dense.T, b_dense.reshape(1, OUT_CH))
    return ans_conv, ans_dense


# speedup vs baseline: 2.2245x; 1.0215x over previous
import functools

import jax
import jax.numpy as jnp
from jax.experimental import pallas as pl
from jax.experimental.pallas import tpu as pltpu
from jax.experimental.pallas import tpu_sc as plsc

N = 8192
IN_CH = 64
OUT_CH = 64
K = 16
EXTENT = 0.1
KS = 4
S = KS * KS * KS

RB = 256
NBLK = N // RB
CW = 8192

SC_NC = 2
SC_NS = 16
SC_NW = SC_NC * SC_NS
GCH = 128
TW = 80


def _atan_poly(t):
    a = jnp.abs(t)
    big = a > 0.4142135623730951
    x = jnp.where(big, (a - 1.0) / (a + 1.0), a)
    z = x * x
    p = (((8.05374449538e-2 * z - 1.38776856032e-1) * z + 1.99777106478e-1) * z
         - 3.33329491539e-1) * z * x + x
    p = jnp.where(big, 0.7853981633974483 + p, p)
    return jnp.sign(t) * p


def _knn_kernel(pos_ref, posT_ref, sq_ref, sqT_ref, idx_ref, d2_ref):
    pid = pl.program_id(0)
    pblk = pos_ref[...]
    sqb = sq_ref[...]
    rid = pid * RB + jax.lax.broadcasted_iota(jnp.int32, (RB, CW), 0)
    cid0 = jax.lax.broadcasted_iota(jnp.int32, (RB, CW), 1)

    def build(c, carry):
        off = pl.multiple_of(c * CW, CW)
        dot = jnp.dot(pblk, posT_ref[:, pl.ds(off, CW)],
                      preferred_element_type=jnp.float32)
        d2 = sqb + sqT_ref[0:1, pl.ds(off, CW)] - 2.0 * dot
        d2_ref[:, pl.ds(off, CW)] = jnp.where(rid == (cid0 + c * CW), jnp.inf, d2)
        return carry

    jax.lax.fori_loop(0, N // CW, build, 0)

    slot = jax.lax.broadcasted_iota(jnp.int32, (RB, K), 1)

    def outer(t, carry):
        lv, li, acc = carry

        def inner(c, ic):
            mv, mi = ic
            off = pl.multiple_of(c * CW, CW)
            d2c = d2_ref[:, pl.ds(off, CW)]
            colc = cid0 + c * CW
            pred = (d2c > lv) | ((d2c == lv) & (colc > li))
            cand = jnp.where(pred, d2c, jnp.inf)
            lmv = jnp.min(cand, axis=1, keepdims=True)
            lmi = jnp.min(jnp.where(cand == lmv, colc, N), axis=1, keepdims=True)
            better = (lmv < mv) | ((lmv == mv) & (lmi < mi))
            return jnp.where(better, lmv, mv), jnp.where(better, lmi, mi)

        mv0 = jnp.full((RB, 1), jnp.inf, jnp.float32)
        mi0 = jnp.full((RB, 1), N, jnp.int32)
        mv, mi = jax.lax.fori_loop(0, N // CW, inner, (mv0, mi0))
        acc = jnp.where(slot == t, mi, acc)
        return mv, mi, acc

    lv0 = jnp.full((RB, 1), -jnp.inf, jnp.float32)
    li0 = jnp.full((RB, 1), -1, jnp.int32)
    acc0 = jnp.zeros((RB, K), jnp.int32)
    _, _, acc = jax.lax.fori_loop(0, K, outer, (lv0, li0, acc0))
    idx_ref[...] = acc


def _knn(pos, posT, sq):
    return pl.pallas_call(
        _knn_kernel,
        grid=(NBLK,),
        in_specs=[
            pl.BlockSpec((RB, 3), lambda i: (i, 0)),
            pl.BlockSpec((3, N), lambda i: (0, 0)),
            pl.BlockSpec((RB, 1), lambda i: (i, 0)),
            pl.BlockSpec((1, N), lambda i: (0, 0)),
        ],
        out_specs=pl.BlockSpec((RB, K), lambda i: (i, 0)),
        out_shape=jax.ShapeDtypeStruct((N, K), jnp.int32),
        scratch_shapes=[pltpu.VMEM((RB, N), jnp.float32)],
    )(pos, posT, sq.reshape(N, 1), sq.reshape(1, N))


def _sc_gather(table, idx_flat):
    b_per_w = (N * K) // SC_NW
    nch = b_per_w // GCH
    mesh = plsc.VectorSubcoreMesh(core_axis_name="c", subcore_axis_name="s")

    @functools.partial(
        pl.kernel,
        mesh=mesh,
        compiler_params=pltpu.CompilerParams(use_tc_tiling_on_sc=False),
        out_type=jax.ShapeDtypeStruct((N * K, TW), jnp.float32),
        scratch_types=[
            pltpu.VMEM((GCH,), jnp.int32),
            pltpu.VMEM((GCH, TW), jnp.float32),
            pltpu.SemaphoreType.DMA,
        ],
    )
    def k(table_hbm, idx_hbm, out_hbm, idx_v, rows_v, sem):
        wid = jax.lax.axis_index("s") * SC_NC + jax.lax.axis_index("c")
        base = wid * b_per_w

        def body(c, carry):
            start = base + c * GCH
            pltpu.sync_copy(idx_hbm.at[pl.ds(start, GCH)], idx_v)
            pltpu.async_copy(table_hbm.at[idx_v], rows_v, sem).wait()
            pltpu.sync_copy(rows_v, out_hbm.at[pl.ds(start, GCH)])
            return carry

        jax.lax.fori_loop(0, nch, body, 0)

    return k(table, idx_flat)


def _conv_kernel(gath_ref, pos_ref, feats_ref, wfr_ref, bc_ref, wdt_ref,
                 bd_ref, conv_ref, dense_ref):
    g = gath_ref[...]
    nbf = g[:, :, 0:IN_CH]
    nbp = g[:, :, IN_CH:IN_CH + 3]
    p = pos_ref[...]
    rel = nbp - p[:, None, :]
    dist2 = jnp.sum(rel * rel, axis=2)
    radius = EXTENT / 2.0
    valid = (dist2 <= radius * radius).astype(jnp.float32)

    rel_n = rel * (2.0 / EXTENT)
    nrm = jnp.sqrt(jnp.sum(rel_n * rel_n, axis=2) + 1e-20)
    scale = jnp.minimum(1.0, 1.0 / nrm)
    x = rel_n[:, :, 0] * scale
    y = rel_n[:, :, 1] * scale
    z = rel_n[:, :, 2] * scale

    sq_norm = x * x + y * y + z * z
    norm = jnp.sqrt(sq_norm + 1e-20)
    sq_xy = x * x + y * y
    cond = (5.0 / 4.0) * z * z > sq_xy
    s1 = jnp.sqrt(3.0 * norm / (norm + jnp.abs(z) + 1e-20))
    x1, y1, z1 = x * s1, y * s1, jnp.sign(z) * norm
    s2 = norm / jnp.sqrt(sq_xy + 1e-20)
    x2, y2, z2 = x * s2, y * s2, 1.5 * z
    x = jnp.where(cond, x1, x2)
    y = jnp.where(cond, y1, y2)
    z = jnp.where(cond, z1, z2)
    nz = sq_norm > 1e-18
    x = jnp.where(nz, x, 0.0)
    y = jnp.where(nz, y, 0.0)
    z = jnp.where(nz, z, 0.0)
    sq_xy2 = x * x + y * y
    norm_xy = jnp.sqrt(sq_xy2 + 1e-20)
    cond2 = jnp.abs(y) <= jnp.abs(x)
    safe_x = jnp.where(jnp.abs(x) > 1e-12, x, 1.0)
    t1 = jnp.where(jnp.abs(x) > 1e-12, y / safe_x, 0.0)
    a1 = jnp.sign(x) * norm_xy
    b1 = jnp.sign(x) * norm_xy * (4.0 / jnp.pi) * _atan_poly(t1)
    safe_y = jnp.where(jnp.abs(y) > 1e-12, y, 1.0)
    t2 = jnp.where(jnp.abs(y) > 1e-12, x / safe_y, 0.0)
    b2 = jnp.sign(y) * norm_xy
    a2 = jnp.sign(y) * norm_xy * (4.0 / jnp.pi) * _atan_poly(t2)
    cx = jnp.where(cond2, a1, a2)
    cy = jnp.where(cond2, b1, b2)
    nz2 = sq_xy2 > 1e-18
    cx = jnp.where(nz2, cx, 0.0)
    cy = jnp.where(nz2, cy, 0.0)
    cz = z

    def axis_grid(cc):
        gg = (cc * 0.5 + 0.5) * (KS - 1)
        gg = jnp.clip(gg, 0.0, KS - 1.0)
        i0f = jnp.clip(jnp.floor(gg), 0.0, KS - 2.0)
        return i0f.astype(jnp.int32), gg - i0f

    i0x, fx = axis_grid(cx)
    i0y, fy = axis_grid(cy)
    i0z, fz = axis_grid(cz)

    sI = jax.lax.broadcasted_iota(jnp.int32, (RB3, K, S), 2)
    izI = sI // (KS * KS)
    iyI = (sI // KS) % KS
    ixI = sI % KS

    def axis_w(aI, i0, f):
        i0e = i0[:, :, None]
        fe = f[:, :, None]
        return (jnp.where(aI == i0e, 1.0 - fe, 0.0)
                + jnp.where(aI == i0e + 1, fe, 0.0))

    w3 = (axis_w(izI, i0z, fz) * axis_w(iyI, i0y, fy) * axis_w(ixI, i0x, fx)
          * valid[:, :, None])

    acc = jax.lax.dot_general(w3, nbf, (((1,), (1,)), ((0,), (0,))),
                              preferred_element_type=jnp.float32,
                              precision=jax.lax.Precision.HIGHEST)
    outS = jax.lax.dot_general(acc, wfr_ref[...], (((2,), (1,)), ((1,), (0,))),
                               preferred_element_type=jnp.float32)
    conv_ref[...] = jnp.sum(outS, axis=0) + bc_ref[...]

    dense_ref[...] = jnp.dot(feats_ref[...], wdt_ref[...],
                             preferred_element_type=jnp.float32) + bd_ref[...]


RB3 = 128
NBLK3 = N // RB3


def _conv(gath, pos, feats, wfr, bc, wdt, bd):
    return pl.pallas_call(
        _conv_kernel,
        grid=(NBLK3,),
        in_specs=[
            pl.BlockSpec((RB3, K, TW), lambda i: (i, 0, 0)),
            pl.BlockSpec((RB3, 3), lambda i: (i, 0)),
            pl.BlockSpec((RB3, IN_CH), lambda i: (i, 0)),
            pl.BlockSpec((S, IN_CH, OUT_CH), lambda i: (0, 0, 0)),
            pl.BlockSpec((1, OUT_CH), lambda i: (0, 0)),
            pl.BlockSpec((IN_CH, OUT_CH), lambda i: (0, 0)),
            pl.BlockSpec((1, OUT_CH), lambda i: (0, 0)),
        ],
        out_specs=[
            pl.BlockSpec((RB3, OUT_CH), lambda i: (i, 0)),
            pl.BlockSpec((RB3, OUT_CH), lambda i: (i, 0)),
        ],
        out_shape=[
            jax.ShapeDtypeStruct((N, OUT_CH), jnp.float32),
            jax.ShapeDtypeStruct((N, OUT_CH), jnp.float32),
        ],
    )(gath, pos, feats, wfr, bc, wdt, bd)


def kernel(feats, pos, W_conv, b_conv, W_dense, b_dense):
    posT = pos.T
    sq = jnp.sum(pos * pos, axis=1)
    idx = _knn(pos, posT, sq)
    table = jnp.concatenate(
        [feats, pos, jnp.zeros((N, TW - IN_CH - 3), jnp.float32)], axis=1)
    gath = _sc_gather(table, idx.reshape(N * K))
    wfr = W_conv.reshape(S, IN_CH, OUT_CH)
    ans_conv, ans_dense = _conv(
        gath.reshape(N, K, TW), pos, feats,
        wfr, b_conv.reshape(1, OUT_CH), W_dense.T, b_dense.reshape(1, OUT_CH))
    return ans_conv, ans_dense
